# baseline (speedup 1.0000x reference)
import functools

import jax
import jax.numpy as jnp
from jax import lax
from jax.experimental import pallas as pl
from jax.experimental.pallas import tpu as pltpu
from jax.experimental.pallas import tpu_sc as plsc

N_NODES = 10000
N_EDGES = 160000
D_IN = 256
D_HID = 200
N_CLASSES = 64

NP = 10240
EP = 163840
DUMMY = 10200
K = 128
D = 128

NC, NS = 2, 16
NW = NC * NS
N_CHUNKS = EP // K
CH0, CH1 = 80, 0
CH_MAX = max(CH0, CH1)
W = 8
R_PER_S = NP // NS
R_CH = R_PER_S // K


NB = 2


def _sc_scatter_kernel(gather):
    mesh = plsc.VectorSubcoreMesh(core_axis_name="c", subcore_axis_name="s")

    scratch = [
        pltpu.VMEM((W, K), jnp.int32),
        pltpu.VMEM((W, K), jnp.int32),
        pltpu.VMEM_SHARED((NP, D), jnp.float32),
    ]
    scratch += [pltpu.VMEM((K, D), jnp.float32) for _ in range(NB)]
    scratch += [pltpu.SemaphoreType.DMA for _ in range(2 * NB)]

    @functools.partial(
        pl.kernel,
        mesh=mesh,
        out_type=jax.ShapeDtypeStruct((NC * NP, D), jnp.float32),
        scratch_types=scratch,
    )
    def body(vals_hbm, src_hbm, dst_hbm, zrows_hbm, out_hbm,
             idx_s, idx_d, acc, *bufs):
        rows = bufs[:NB]
        gsem = bufs[NB:2 * NB]
        ssem = bufs[2 * NB:]
        cid = lax.axis_index("c")
        sid = lax.axis_index("s")

        c0 = pl.multiple_of(
            lax.select(cid == 0, sid * CH0, NS * CH0 + sid * CH1), 8)
        n_w = lax.select(cid == 0, CH0 // W, CH1 // W)

        pltpu.sync_copy(zrows_hbm, rows[0])
        row0 = pl.multiple_of(sid * R_PER_S, K)

        def zero_body(j, carry):
            b = pl.multiple_of(row0 + j * K, K)
            pltpu.sync_copy(rows[0], acc.at[pl.ds(b, K)])
            return carry

        lax.fori_loop(0, R_CH, zero_body, 0)
        if not gather:
            pltpu.sync_copy(vals_hbm, rows[0])
        plsc.subcore_barrier()

        def win_body(w, carry):
            cb = pl.multiple_of(c0 + w * W, 8)
            pltpu.sync_copy(dst_hbm.at[pl.ds(cb, W)], idx_d)
            if gather:
                pltpu.sync_copy(src_hbm.at[pl.ds(cb, W)], idx_s)

            def edge_body(t, carry2):
                base = t * NB
                if gather:
                    gs = [pltpu.async_copy(vals_hbm.at[idx_s.at[base + b]],
                                           rows[b], gsem[b])
                          for b in range(NB)]
                    ss = []
                    for b in range(NB):
                        gs[b].wait()
                        ss.append(pltpu.async_copy(
                            rows[b], acc.at[idx_d.at[base + b]], ssem[b],
                            add=True))
                    for s in ss:
                        s.wait()
                else:
                    ss = [pltpu.async_copy(rows[0],
                                           acc.at[idx_d.at[base + b]],
                                           ssem[b], add=True)
                          for b in range(NB)]
                    for s in ss:
                        s.wait()
                return carry2

            lax.fori_loop(0, W // NB, edge_body, 0)
            return carry

        lax.fori_loop(0, n_w, win_body, 0)
        plsc.subcore_barrier()

        o0 = pl.multiple_of(cid * NP + sid * R_PER_S, K)

        def a_sl(j):
            return pl.ds(pl.multiple_of(sid * R_PER_S + j * K, K), K)

        def o_sl(j):
            return pl.ds(pl.multiple_of(o0 + j * K, K), K)

        loads = [pltpu.async_copy(acc.at[a_sl(j)], rows[j], gsem[j])
                 for j in range(NB)]
        outs = []
        for j in range(NB):
            loads[j].wait()
            outs.append(pltpu.async_copy(rows[j], out_hbm.at[o_sl(j)],
                                         ssem[j]))
        for j in range(NB, R_CH):
            b = j % NB
            outs[j - NB].wait()
            pltpu.async_copy(acc.at[a_sl(j)], rows[b], gsem[b]).wait()
            outs.append(pltpu.async_copy(rows[b], out_hbm.at[o_sl(j)],
                                         ssem[b]))
        for o in outs[max(0, R_CH - NB):]:
            o.wait()

    return body


def _dinv(degp_ref):
    deg = degp_ref[0, :, 0:1] + degp_ref[1, :, 0:1] + 1.0
    return lax.rsqrt(deg)


def _tc1_body(x_ref, w_ref, degp_ref, oa_ref, ob_ref):
    dinv = _dinv(degp_ref)
    h = jnp.dot(x_ref[...], w_ref[...], preferred_element_type=jnp.float32)
    hs = h * dinv
    oa_ref[...] = hs[:, :D]
    ob_ref[...] = hs[:, D:]


def _tc2_body(acca_ref, accb_ref, hsa_ref, hsb_ref, degp_ref, b_ref, w_ref,
              o_ref):
    dinv = _dinv(degp_ref)
    xa = acca_ref[0] + acca_ref[1] + hsa_ref[...]
    xb = accb_ref[0] + accb_ref[1] + hsb_ref[...]
    x1 = jnp.concatenate([xa, xb], axis=1)
    x1 = jnp.maximum(x1 * dinv + b_ref[...], 0.0)
    h2 = jnp.dot(x1, w_ref[...], preferred_element_type=jnp.float32)
    o_ref[...] = h2 * dinv


def _tc3_body(accp_ref, hs_ref, degp_ref, b_ref, o_ref):
    dinv = _dinv(degp_ref)
    x = accp_ref[0] + accp_ref[1] + hs_ref[...]
    o_ref[...] = jnp.maximum(x * dinv + b_ref[...], 0.0)


_RB = 1024


def kernel(features, edge_index, W1, b1, W2, b2):
    f32 = jnp.float32
    x = jnp.pad(features.astype(f32), ((0, NP - N_NODES), (0, 0)))
    src = edge_index[0].astype(jnp.int32)
    dst = edge_index[1].astype(jnp.int32)
    pad_e = (N_CHUNKS + CH_MAX) * K - N_EDGES
    src = jnp.concatenate([src, jnp.full((pad_e,), DUMMY, jnp.int32)])
    dst = jnp.concatenate([dst, jnp.full((pad_e,), DUMMY, jnp.int32)])
    src = src.reshape(-1, K)
    dst = dst.reshape(-1, K)

    zrows = jnp.zeros((K, D), f32)
    ones = jnp.ones((K, D), f32)
    w1p = jnp.pad(W1.astype(f32), ((0, 0), (0, 2 * D - D_HID)))
    b1p = jnp.pad(b1.astype(f32), (0, 2 * D - D_HID))[None, :]
    w2p = jnp.pad(W2.astype(f32), ((0, 2 * D - D_HID), (0, D - N_CLASSES)))
    b2p = jnp.pad(b2.astype(f32), (0, D - N_CLASSES))[None, :]

    degp = _sc_scatter_kernel(False)(ones, src, dst, zrows)
    degp = degp.reshape(NC, NP, D)

    degp_spec = pl.BlockSpec((NC, _RB, D), lambda i: (0, i, 0))
    row_spec = pl.BlockSpec((_RB, D), lambda i: (i, 0))
    acc_spec = pl.BlockSpec((NC, _RB, D), lambda i: (0, i, 0))

    hs1a, hs1b = pl.pallas_call(
        _tc1_body,
        grid=(NP // _RB,),
        in_specs=[pl.BlockSpec((_RB, D_IN), lambda i: (i, 0)),
                  pl.BlockSpec((D_IN, 2 * D), lambda i: (0, 0)),
                  degp_spec],
        out_specs=[row_spec, row_spec],
        out_shape=[jax.ShapeDtypeStruct((NP, D), f32),
                   jax.ShapeDtypeStruct((NP, D), f32)],
    )(x, w1p, degp)

    sc_gather = _sc_scatter_kernel(True)
    acc1a = sc_gather(hs1a, src, dst, zrows).reshape(NC, NP, D)
    acc1b = sc_gather(hs1b, src, dst, zrows).reshape(NC, NP, D)

    hs2 = pl.pallas_call(
        _tc2_body,
        grid=(NP // _RB,),
        in_specs=[acc_spec, acc_spec, row_spec, row_spec, degp_spec,
                  pl.BlockSpec((1, 2 * D), lambda i: (0, 0)),
                  pl.BlockSpec((2 * D, D), lambda i: (0, 0))],
        out_specs=row_spec,
        out_shape=jax.ShapeDtypeStruct((NP, D), f32),
    )(acc1a, acc1b, hs1a, hs1b, degp, b1p, w2p)

    acc2 = sc_gather(hs2, src, dst, zrows).reshape(NC, NP, D)

    out = pl.pallas_call(
        _tc3_body,
        grid=(NP // _RB,),
        in_specs=[acc_spec, row_spec, degp_spec,
                  pl.BlockSpec((1, D), lambda i: (0, 0))],
        out_specs=row_spec,
        out_shape=jax.ShapeDtypeStruct((NP, D), f32),
    )(acc2, hs2, degp, b2p)

    return out[:N_NODES, :N_CLASSES]


# speedup vs baseline: 1.4451x; 1.4451x over previous
import functools

import jax
import jax.numpy as jnp
from jax import lax
from jax.experimental import pallas as pl
from jax.experimental.pallas import tpu as pltpu
from jax.experimental.pallas import tpu_sc as plsc

N_NODES = 10000
N_EDGES = 160000
D_IN = 256
D_HID = 200
N_CLASSES = 64

NP = 10240
EP = 163840
DUMMY = 10200
K = 128
D = 128

NC, NS = 2, 16
NW = NC * NS
N_CHUNKS = EP // K
CH0, CH1 = 64, 16
CH_MAX = max(CH0, CH1)
W = 8
R_PER_S = NP // NS
R_CH = R_PER_S // K


NB = 2


def _sc_scatter_kernel(gather):
    mesh = plsc.VectorSubcoreMesh(core_axis_name="c", subcore_axis_name="s")

    scratch = [
        pltpu.VMEM((W, K), jnp.int32),
        pltpu.VMEM((W, K), jnp.int32),
        pltpu.VMEM_SHARED((NP, D), jnp.float32),
    ]
    scratch += [pltpu.VMEM((K, D), jnp.float32) for _ in range(NB)]
    scratch += [pltpu.SemaphoreType.DMA for _ in range(2 * NB)]

    @functools.partial(
        pl.kernel,
        mesh=mesh,
        out_type=jax.ShapeDtypeStruct((NC * NP, D), jnp.float32),
        scratch_types=scratch,
    )
    def body(vals_hbm, src_hbm, dst_hbm, zrows_hbm, out_hbm,
             idx_s, idx_d, acc, *bufs):
        rows = bufs[:NB]
        gsem = bufs[NB:2 * NB]
        ssem = bufs[2 * NB:]
        cid = lax.axis_index("c")
        sid = lax.axis_index("s")

        c0 = pl.multiple_of(
            lax.select(cid == 0, sid * CH0, NS * CH0 + sid * CH1), 8)
        n_w = lax.select(cid == 0, CH0 // W, CH1 // W)

        pltpu.sync_copy(zrows_hbm, rows[0])
        row0 = pl.multiple_of(sid * R_PER_S, K)

        def zero_body(j, carry):
            b = pl.multiple_of(row0 + j * K, K)
            pltpu.sync_copy(rows[0], acc.at[pl.ds(b, K)])
            return carry

        lax.fori_loop(0, R_CH, zero_body, 0)
        if not gather:
            pltpu.sync_copy(vals_hbm, rows[0])
        plsc.subcore_barrier()

        def win_body(w, carry):
            cb = pl.multiple_of(c0 + w * W, 8)
            pltpu.sync_copy(dst_hbm.at[pl.ds(cb, W)], idx_d)
            if gather:
                pltpu.sync_copy(src_hbm.at[pl.ds(cb, W)], idx_s)

            def edge_body(t, carry2):
                base = t * NB
                if gather:
                    gs = [pltpu.async_copy(vals_hbm.at[idx_s.at[base + b]],
                                           rows[b], gsem[b])
                          for b in range(NB)]
                    ss = []
                    for b in range(NB):
                        gs[b].wait()
                        ss.append(pltpu.async_copy(
                            rows[b], acc.at[idx_d.at[base + b]], ssem[b],
                            add=True))
                    for s in ss:
                        s.wait()
                else:
                    ss = [pltpu.async_copy(rows[0],
                                           acc.at[idx_d.at[base + b]],
                                           ssem[b], add=True)
                          for b in range(NB)]
                    for s in ss:
                        s.wait()
                return carry2

            lax.fori_loop(0, W // NB, edge_body, 0)
            return carry

        lax.fori_loop(0, n_w, win_body, 0)
        plsc.subcore_barrier()

        o0 = pl.multiple_of(cid * NP + sid * R_PER_S, K)

        def a_sl(j):
            return pl.ds(pl.multiple_of(sid * R_PER_S + j * K, K), K)

        def o_sl(j):
            return pl.ds(pl.multiple_of(o0 + j * K, K), K)

        loads = [pltpu.async_copy(acc.at[a_sl(j)], rows[j], gsem[j])
                 for j in range(NB)]
        outs = []
        for j in range(NB):
            loads[j].wait()
            outs.append(pltpu.async_copy(rows[j], out_hbm.at[o_sl(j)],
                                         ssem[j]))
        for j in range(NB, R_CH):
            b = j % NB
            outs[j - NB].wait()
            pltpu.async_copy(acc.at[a_sl(j)], rows[b], gsem[b]).wait()
            outs.append(pltpu.async_copy(rows[b], out_hbm.at[o_sl(j)],
                                         ssem[b]))
        for o in outs[max(0, R_CH - NB):]:
            o.wait()

    return body


def _dinv(degp_ref):
    deg = degp_ref[0, :, 0:1] + degp_ref[1, :, 0:1] + 1.0
    return lax.rsqrt(deg)


def _tc1_body(x_ref, w_ref, degp_ref, oa_ref, ob_ref):
    dinv = _dinv(degp_ref)
    h = jnp.dot(x_ref[...], w_ref[...], preferred_element_type=jnp.float32)
    hs = h * dinv
    oa_ref[...] = hs[:, :D]
    ob_ref[...] = hs[:, D:]


def _tc2_body(acca_ref, accb_ref, hsa_ref, hsb_ref, degp_ref, b_ref, w_ref,
              o_ref):
    dinv = _dinv(degp_ref)
    xa = acca_ref[0] + acca_ref[1] + hsa_ref[...]
    xb = accb_ref[0] + accb_ref[1] + hsb_ref[...]
    x1 = jnp.concatenate([xa, xb], axis=1)
    x1 = jnp.maximum(x1 * dinv + b_ref[...], 0.0)
    h2 = jnp.dot(x1, w_ref[...], preferred_element_type=jnp.float32)
    o_ref[...] = h2 * dinv


def _tc3_body(accp_ref, hs_ref, degp_ref, b_ref, o_ref):
    dinv = _dinv(degp_ref)
    x = accp_ref[0] + accp_ref[1] + hs_ref[...]
    o_ref[...] = jnp.maximum(x * dinv + b_ref[...], 0.0)


_RB = 1024


def kernel(features, edge_index, W1, b1, W2, b2):
    f32 = jnp.float32
    x = jnp.pad(features.astype(f32), ((0, NP - N_NODES), (0, 0)))
    src = edge_index[0].astype(jnp.int32)
    dst = edge_index[1].astype(jnp.int32)
    pad_e = (N_CHUNKS + CH_MAX) * K - N_EDGES
    src = jnp.concatenate([src, jnp.full((pad_e,), DUMMY, jnp.int32)])
    dst = jnp.concatenate([dst, jnp.full((pad_e,), DUMMY, jnp.int32)])
    src = src.reshape(-1, K)
    dst = dst.reshape(-1, K)

    zrows = jnp.zeros((K, D), f32)
    ones = jnp.ones((K, D), f32)
    w1p = jnp.pad(W1.astype(f32), ((0, 0), (0, 2 * D - D_HID)))
    b1p = jnp.pad(b1.astype(f32), (0, 2 * D - D_HID))[None, :]
    w2p = jnp.pad(W2.astype(f32), ((0, 2 * D - D_HID), (0, D - N_CLASSES)))
    b2p = jnp.pad(b2.astype(f32), (0, D - N_CLASSES))[None, :]

    degp = _sc_scatter_kernel(False)(ones, src, dst, zrows)
    degp = degp.reshape(NC, NP, D)

    degp_spec = pl.BlockSpec((NC, _RB, D), lambda i: (0, i, 0))
    row_spec = pl.BlockSpec((_RB, D), lambda i: (i, 0))
    acc_spec = pl.BlockSpec((NC, _RB, D), lambda i: (0, i, 0))

    hs1a, hs1b = pl.pallas_call(
        _tc1_body,
        grid=(NP // _RB,),
        in_specs=[pl.BlockSpec((_RB, D_IN), lambda i: (i, 0)),
                  pl.BlockSpec((D_IN, 2 * D), lambda i: (0, 0)),
                  degp_spec],
        out_specs=[row_spec, row_spec],
        out_shape=[jax.ShapeDtypeStruct((NP, D), f32),
                   jax.ShapeDtypeStruct((NP, D), f32)],
    )(x, w1p, degp)

    sc_gather = _sc_scatter_kernel(True)
    acc1a = sc_gather(hs1a, src, dst, zrows).reshape(NC, NP, D)
    acc1b = sc_gather(hs1b, src, dst, zrows).reshape(NC, NP, D)

    hs2 = pl.pallas_call(
        _tc2_body,
        grid=(NP // _RB,),
        in_specs=[acc_spec, acc_spec, row_spec, row_spec, degp_spec,
                  pl.BlockSpec((1, 2 * D), lambda i: (0, 0)),
                  pl.BlockSpec((2 * D, D), lambda i: (0, 0))],
        out_specs=row_spec,
        out_shape=jax.ShapeDtypeStruct((NP, D), f32),
    )(acc1a, acc1b, hs1a, hs1b, degp, b1p, w2p)

    acc2 = sc_gather(hs2, src, dst, zrows).reshape(NC, NP, D)

    out = pl.pallas_call(
        _tc3_body,
        grid=(NP // _RB,),
        in_specs=[acc_spec, row_spec, degp_spec,
                  pl.BlockSpec((1, D), lambda i: (0, 0))],
        out_specs=row_spec,
        out_shape=jax.ShapeDtypeStruct((NP, D), f32),
    )(acc2, hs2, degp, b2p)

    return out[:N_NODES, :N_CLASSES]


# speedup vs baseline: 1.5707x; 1.0869x over previous
import functools

import jax
import jax.numpy as jnp
from jax import lax
from jax.experimental import pallas as pl
from jax.experimental.pallas import tpu as pltpu
from jax.experimental.pallas import tpu_sc as plsc

N_NODES = 10000
N_EDGES = 160000
D_IN = 256
D_HID = 200
N_CLASSES = 64

NP = 10240
EP = 163840
DUMMY = 10200
K = 128
D = 128

NC, NS = 2, 16
NW = NC * NS
N_CHUNKS = EP // K
CH0, CH1 = 76, 4
CH_MAX = max(CH0, CH1)
W = 4
R_PER_S = NP // NS
R_CH = R_PER_S // K


NB = 2


def _sc_scatter_kernel(gather):
    mesh = plsc.VectorSubcoreMesh(core_axis_name="c", subcore_axis_name="s")

    scratch = [
        pltpu.VMEM((W, K), jnp.int32),
        pltpu.VMEM((W, K), jnp.int32),
        pltpu.VMEM_SHARED((NP, D), jnp.float32),
    ]
    scratch += [pltpu.VMEM((K, D), jnp.float32) for _ in range(NB)]
    scratch += [pltpu.SemaphoreType.DMA for _ in range(2 * NB)]

    @functools.partial(
        pl.kernel,
        mesh=mesh,
        out_type=jax.ShapeDtypeStruct((NC * NP, D), jnp.float32),
        scratch_types=scratch,
    )
    def body(vals_hbm, src_hbm, dst_hbm, zrows_hbm, out_hbm,
             idx_s, idx_d, acc, *bufs):
        rows = bufs[:NB]
        gsem = bufs[NB:2 * NB]
        ssem = bufs[2 * NB:]
        cid = lax.axis_index("c")
        sid = lax.axis_index("s")

        c0 = pl.multiple_of(
            lax.select(cid == 0, sid * CH0, NS * CH0 + sid * CH1), 8)
        n_w = lax.select(cid == 0, CH0 // W, CH1 // W)

        pltpu.sync_copy(zrows_hbm, rows[0])
        row0 = pl.multiple_of(sid * R_PER_S, K)

        def zero_body(j, carry):
            b = pl.multiple_of(row0 + j * K, K)
            pltpu.sync_copy(rows[0], acc.at[pl.ds(b, K)])
            return carry

        lax.fori_loop(0, R_CH, zero_body, 0)
        if not gather:
            pltpu.sync_copy(vals_hbm, rows[0])
        plsc.subcore_barrier()

        def win_body(w, carry):
            cb = pl.multiple_of(c0 + w * W, 8)
            pltpu.sync_copy(dst_hbm.at[pl.ds(cb, W)], idx_d)
            if gather:
                pltpu.sync_copy(src_hbm.at[pl.ds(cb, W)], idx_s)

            def edge_body(t, carry2):
                base = t * NB
                if gather:
                    gs = [pltpu.async_copy(vals_hbm.at[idx_s.at[base + b]],
                                           rows[b], gsem[b])
                          for b in range(NB)]
                    ss = []
                    for b in range(NB):
                        gs[b].wait()
                        ss.append(pltpu.async_copy(
                            rows[b], acc.at[idx_d.at[base + b]], ssem[b],
                            add=True))
                    for s in ss:
                        s.wait()
                else:
                    ss = [pltpu.async_copy(rows[0],
                                           acc.at[idx_d.at[base + b]],
                                           ssem[b], add=True)
                          for b in range(NB)]
                    for s in ss:
                        s.wait()
                return carry2

            lax.fori_loop(0, W // NB, edge_body, 0)
            return carry

        lax.fori_loop(0, n_w, win_body, 0)
        plsc.subcore_barrier()

        o0 = pl.multiple_of(cid * NP + sid * R_PER_S, K)

        def a_sl(j):
            return pl.ds(pl.multiple_of(sid * R_PER_S + j * K, K), K)

        def o_sl(j):
            return pl.ds(pl.multiple_of(o0 + j * K, K), K)

        loads = [pltpu.async_copy(acc.at[a_sl(j)], rows[j], gsem[j])
                 for j in range(NB)]
        outs = []
        for j in range(NB):
            loads[j].wait()
            outs.append(pltpu.async_copy(rows[j], out_hbm.at[o_sl(j)],
                                         ssem[j]))
        for j in range(NB, R_CH):
            b = j % NB
            outs[j - NB].wait()
            pltpu.async_copy(acc.at[a_sl(j)], rows[b], gsem[b]).wait()
            outs.append(pltpu.async_copy(rows[b], out_hbm.at[o_sl(j)],
                                         ssem[b]))
        for o in outs[max(0, R_CH - NB):]:
            o.wait()

    return body


def _dinv(degp_ref):
    deg = degp_ref[0, :, 0:1] + degp_ref[1, :, 0:1] + 1.0
    return lax.rsqrt(deg)


def _tc1_body(x_ref, w_ref, degp_ref, oa_ref, ob_ref):
    dinv = _dinv(degp_ref)
    h = jnp.dot(x_ref[...], w_ref[...], preferred_element_type=jnp.float32)
    hs = h * dinv
    oa_ref[...] = hs[:, :D]
    ob_ref[...] = hs[:, D:]


def _tc2_body(acca_ref, accb_ref, hsa_ref, hsb_ref, degp_ref, b_ref, w_ref,
              o_ref):
    dinv = _dinv(degp_ref)
    xa = acca_ref[0] + acca_ref[1] + hsa_ref[...]
    xb = accb_ref[0] + accb_ref[1] + hsb_ref[...]
    x1 = jnp.concatenate([xa, xb], axis=1)
    x1 = jnp.maximum(x1 * dinv + b_ref[...], 0.0)
    h2 = jnp.dot(x1, w_ref[...], preferred_element_type=jnp.float32)
    o_ref[...] = h2 * dinv


def _tc3_body(accp_ref, hs_ref, degp_ref, b_ref, o_ref):
    dinv = _dinv(degp_ref)
    x = accp_ref[0] + accp_ref[1] + hs_ref[...]
    o_ref[...] = jnp.maximum(x * dinv + b_ref[...], 0.0)


_RB = 1024


def kernel(features, edge_index, W1, b1, W2, b2):
    f32 = jnp.float32
    x = jnp.pad(features.astype(f32), ((0, NP - N_NODES), (0, 0)))
    src = edge_index[0].astype(jnp.int32)
    dst = edge_index[1].astype(jnp.int32)
    pad_e = (N_CHUNKS + CH_MAX) * K - N_EDGES
    src = jnp.concatenate([src, jnp.full((pad_e,), DUMMY, jnp.int32)])
    dst = jnp.concatenate([dst, jnp.full((pad_e,), DUMMY, jnp.int32)])
    src = src.reshape(-1, K)
    dst = dst.reshape(-1, K)

    zrows = jnp.zeros((K, D), f32)
    ones = jnp.ones((K, D), f32)
    w1p = jnp.pad(W1.astype(f32), ((0, 0), (0, 2 * D - D_HID)))
    b1p = jnp.pad(b1.astype(f32), (0, 2 * D - D_HID))[None, :]
    w2p = jnp.pad(W2.astype(f32), ((0, 2 * D - D_HID), (0, D - N_CLASSES)))
    b2p = jnp.pad(b2.astype(f32), (0, D - N_CLASSES))[None, :]

    degp = _sc_scatter_kernel(False)(ones, src, dst, zrows)
    degp = degp.reshape(NC, NP, D)

    degp_spec = pl.BlockSpec((NC, _RB, D), lambda i: (0, i, 0))
    row_spec = pl.BlockSpec((_RB, D), lambda i: (i, 0))
    acc_spec = pl.BlockSpec((NC, _RB, D), lambda i: (0, i, 0))

    hs1a, hs1b = pl.pallas_call(
        _tc1_body,
        grid=(NP // _RB,),
        in_specs=[pl.BlockSpec((_RB, D_IN), lambda i: (i, 0)),
                  pl.BlockSpec((D_IN, 2 * D), lambda i: (0, 0)),
                  degp_spec],
        out_specs=[row_spec, row_spec],
        out_shape=[jax.ShapeDtypeStruct((NP, D), f32),
                   jax.ShapeDtypeStruct((NP, D), f32)],
    )(x, w1p, degp)

    sc_gather = _sc_scatter_kernel(True)
    acc1a = sc_gather(hs1a, src, dst, zrows).reshape(NC, NP, D)
    acc1b = sc_gather(hs1b, src, dst, zrows).reshape(NC, NP, D)

    hs2 = pl.pallas_call(
        _tc2_body,
        grid=(NP // _RB,),
        in_specs=[acc_spec, acc_spec, row_spec, row_spec, degp_spec,
                  pl.BlockSpec((1, 2 * D), lambda i: (0, 0)),
                  pl.BlockSpec((2 * D, D), lambda i: (0, 0))],
        out_specs=row_spec,
        out_shape=jax.ShapeDtypeStruct((NP, D), f32),
    )(acc1a, acc1b, hs1a, hs1b, degp, b1p, w2p)

    acc2 = sc_gather(hs2, src, dst, zrows).reshape(NC, NP, D)

    out = pl.pallas_call(
        _tc3_body,
        grid=(NP // _RB,),
        in_specs=[acc_spec, row_spec, degp_spec,
                  pl.BlockSpec((1, D), lambda i: (0, 0))],
        out_specs=row_spec,
        out_shape=jax.ShapeDtypeStruct((NP, D), f32),
    )(acc2, hs2, degp, b2p)

    return out[:N_NODES, :N_CLASSES]


# speedup vs baseline: 1.6713x; 1.0640x over previous
import functools

import jax
import jax.numpy as jnp
from jax import lax
from jax.experimental import pallas as pl
from jax.experimental.pallas import tpu as pltpu
from jax.experimental.pallas import tpu_sc as plsc

N_NODES = 10000
N_EDGES = 160000
D_IN = 256
D_HID = 200
N_CLASSES = 64

NP = 10240
EP = 163840
DUMMY = 10200
K = 128
D = 128

NC, NS = 2, 16
NW = NC * NS
N_CHUNKS = EP // K
CH0, CH1 = 76, 4
CH_MAX = max(CH0, CH1)
W = 4
R_PER_S = NP // NS
R_CH = R_PER_S // K


NB = 2


def _sc_scatter_kernel(gather, d=D):
    mesh = plsc.VectorSubcoreMesh(core_axis_name="c", subcore_axis_name="s")

    scratch = [
        pltpu.VMEM((W, K), jnp.int32),
        pltpu.VMEM((W, K), jnp.int32),
        pltpu.VMEM_SHARED((NP, d), jnp.float32),
    ]
    scratch += [pltpu.VMEM((K, d), jnp.float32) for _ in range(NB)]
    scratch += [pltpu.SemaphoreType.DMA for _ in range(2 * NB)]

    @functools.partial(
        pl.kernel,
        mesh=mesh,
        out_type=jax.ShapeDtypeStruct((NC * NP, d), jnp.float32),
        scratch_types=scratch,
    )
    def body(vals_hbm, src_hbm, dst_hbm, zrows_hbm, out_hbm,
             idx_s, idx_d, acc, *bufs):
        rows = bufs[:NB]
        gsem = bufs[NB:2 * NB]
        ssem = bufs[2 * NB:]
        cid = lax.axis_index("c")
        sid = lax.axis_index("s")

        c0 = pl.multiple_of(
            lax.select(cid == 0, sid * CH0, NS * CH0 + sid * CH1), 8)
        n_w = lax.select(cid == 0, CH0 // W, CH1 // W)

        pltpu.sync_copy(zrows_hbm, rows[0])
        row0 = pl.multiple_of(sid * R_PER_S, K)

        def zero_body(j, carry):
            b = pl.multiple_of(row0 + j * K, K)
            pltpu.sync_copy(rows[0], acc.at[pl.ds(b, K)])
            return carry

        lax.fori_loop(0, R_CH, zero_body, 0)
        if not gather:
            pltpu.sync_copy(vals_hbm, rows[0])
        plsc.subcore_barrier()

        def win_body(w, carry):
            cb = pl.multiple_of(c0 + w * W, 8)
            pltpu.sync_copy(dst_hbm.at[pl.ds(cb, W)], idx_d)
            if gather:
                pltpu.sync_copy(src_hbm.at[pl.ds(cb, W)], idx_s)

            def edge_body(t, carry2):
                base = t * NB
                if gather:
                    gs = [pltpu.async_copy(vals_hbm.at[idx_s.at[base + b]],
                                           rows[b], gsem[b])
                          for b in range(NB)]
                    ss = []
                    for b in range(NB):
                        gs[b].wait()
                        ss.append(pltpu.async_copy(
                            rows[b], acc.at[idx_d.at[base + b]], ssem[b],
                            add=True))
                    for s in ss:
                        s.wait()
                else:
                    ss = [pltpu.async_copy(rows[0],
                                           acc.at[idx_d.at[base + b]],
                                           ssem[b], add=True)
                          for b in range(NB)]
                    for s in ss:
                        s.wait()
                return carry2

            lax.fori_loop(0, W // NB, edge_body, 0)
            return carry

        lax.fori_loop(0, n_w, win_body, 0)
        plsc.subcore_barrier()

        o0 = pl.multiple_of(cid * NP + sid * R_PER_S, K)

        def a_sl(j):
            return pl.ds(pl.multiple_of(sid * R_PER_S + j * K, K), K)

        def o_sl(j):
            return pl.ds(pl.multiple_of(o0 + j * K, K), K)

        loads = [pltpu.async_copy(acc.at[a_sl(j)], rows[j], gsem[j])
                 for j in range(NB)]
        outs = []
        for j in range(NB):
            loads[j].wait()
            outs.append(pltpu.async_copy(rows[j], out_hbm.at[o_sl(j)],
                                         ssem[j]))
        for j in range(NB, R_CH):
            b = j % NB
            outs[j - NB].wait()
            pltpu.async_copy(acc.at[a_sl(j)], rows[b], gsem[b]).wait()
            outs.append(pltpu.async_copy(rows[b], out_hbm.at[o_sl(j)],
                                         ssem[b]))
        for o in outs[max(0, R_CH - NB):]:
            o.wait()

    return body


def _dinv(degp_ref):
    deg = degp_ref[0, :, 0:1] + degp_ref[1, :, 0:1] + 1.0
    return lax.rsqrt(deg)


def _tc1_body(x_ref, w_ref, degp_ref, oa_ref, ob_ref):
    dinv = _dinv(degp_ref)
    h = jnp.dot(x_ref[...], w_ref[...], preferred_element_type=jnp.float32)
    hs = h * dinv
    oa_ref[...] = hs[:, :D]
    ob_ref[...] = hs[:, D:]


def _tc2_body(acca_ref, accb_ref, hsa_ref, hsb_ref, degp_ref, b_ref, w_ref,
              o_ref):
    dinv = _dinv(degp_ref)
    xa = acca_ref[0] + acca_ref[1] + hsa_ref[...]
    xb = accb_ref[0] + accb_ref[1] + hsb_ref[...]
    x1 = jnp.concatenate([xa, xb], axis=1)
    x1 = jnp.maximum(x1 * dinv + b_ref[...], 0.0)
    h2 = jnp.dot(x1, w_ref[...], preferred_element_type=jnp.float32)
    o_ref[...] = h2 * dinv


def _tc3_body(accp_ref, hs_ref, degp_ref, b_ref, o_ref):
    dinv = _dinv(degp_ref)
    x = accp_ref[0] + accp_ref[1] + hs_ref[...]
    o_ref[...] = jnp.maximum(x * dinv + b_ref[...], 0.0)


_RB = 1024


def kernel(features, edge_index, W1, b1, W2, b2):
    f32 = jnp.float32
    x = jnp.pad(features.astype(f32), ((0, NP - N_NODES), (0, 0)))
    src = edge_index[0].astype(jnp.int32)
    dst = edge_index[1].astype(jnp.int32)
    pad_e = (N_CHUNKS + CH_MAX) * K - N_EDGES
    src = jnp.concatenate([src, jnp.full((pad_e,), DUMMY, jnp.int32)])
    dst = jnp.concatenate([dst, jnp.full((pad_e,), DUMMY, jnp.int32)])
    src = src.reshape(-1, K)
    dst = dst.reshape(-1, K)

    zrows = jnp.zeros((K, D), f32)
    DW = 8
    zrows_d = jnp.zeros((K, DW), f32)
    ones = jnp.ones((K, DW), f32)
    w1p = jnp.pad(W1.astype(f32), ((0, 0), (0, 2 * D - D_HID)))
    b1p = jnp.pad(b1.astype(f32), (0, 2 * D - D_HID))[None, :]
    w2p = jnp.pad(W2.astype(f32), ((0, 2 * D - D_HID), (0, D - N_CLASSES)))
    b2p = jnp.pad(b2.astype(f32), (0, D - N_CLASSES))[None, :]

    degp = _sc_scatter_kernel(False, 8)(ones, src, dst, zrows_d)
    degp = degp.reshape(NC, NP, DW)

    degp_spec = pl.BlockSpec((NC, _RB, DW), lambda i: (0, i, 0))
    row_spec = pl.BlockSpec((_RB, D), lambda i: (i, 0))
    acc_spec = pl.BlockSpec((NC, _RB, D), lambda i: (0, i, 0))

    hs1a, hs1b = pl.pallas_call(
        _tc1_body,
        grid=(NP // _RB,),
        in_specs=[pl.BlockSpec((_RB, D_IN), lambda i: (i, 0)),
                  pl.BlockSpec((D_IN, 2 * D), lambda i: (0, 0)),
                  degp_spec],
        out_specs=[row_spec, row_spec],
        out_shape=[jax.ShapeDtypeStruct((NP, D), f32),
                   jax.ShapeDtypeStruct((NP, D), f32)],
    )(x, w1p, degp)

    sc_gather = _sc_scatter_kernel(True)
    acc1a = sc_gather(hs1a, src, dst, zrows).reshape(NC, NP, D)
    acc1b = sc_gather(hs1b, src, dst, zrows).reshape(NC, NP, D)

    hs2 = pl.pallas_call(
        _tc2_body,
        grid=(NP // _RB,),
        in_specs=[acc_spec, acc_spec, row_spec, row_spec, degp_spec,
                  pl.BlockSpec((1, 2 * D), lambda i: (0, 0)),
                  pl.BlockSpec((2 * D, D), lambda i: (0, 0))],
        out_specs=row_spec,
        out_shape=jax.ShapeDtypeStruct((NP, D), f32),
    )(acc1a, acc1b, hs1a, hs1b, degp, b1p, w2p)

    acc2 = sc_gather(hs2, src, dst, zrows).reshape(NC, NP, D)

    out = pl.pallas_call(
        _tc3_body,
        grid=(NP // _RB,),
        in_specs=[acc_spec, row_spec, degp_spec,
                  pl.BlockSpec((1, D), lambda i: (0, 0))],
        out_specs=row_spec,
        out_shape=jax.ShapeDtypeStruct((NP, D), f32),
    )(acc2, hs2, degp, b2p)

    return out[:N_NODES, :N_CLASSES]
